# baseline (speedup 1.0000x reference)
import jax
import jax.numpy as jnp
import numpy as np
from jax.experimental import pallas as pl
from jax.experimental.pallas import tpu as pltpu

N_BINS = 15


def _bin_bounds():
    lane_i = jax.lax.broadcasted_iota(jnp.int32, (1, 128), 1)
    lane = lane_i.astype(jnp.float32)
    valid = lane_i < N_BINS
    lowers = jnp.where(valid, lane / N_BINS, 2.0)
    uppers = jnp.where(valid, (lane + 1.0) / N_BINS, 3.0)
    return lowers, uppers


def _ece_kernel(logits_ref, labels_ref, out_ref, acc_ref, *, nblocks, n_total):
    i = pl.program_id(0)

    @pl.when(i == 0)
    def _init():
        acc_ref[...] = jnp.zeros_like(acc_ref)

    x = logits_ref[...]
    m = jnp.max(x, axis=1, keepdims=True)
    s = jnp.sum(jnp.exp(x - m), axis=1, keepdims=True)
    conf = 1.0 / s
    pred = jnp.argmax(x, axis=1).reshape(-1, 1)
    correct = (pred == labels_ref[...]).astype(jnp.float32)

    lowers, uppers = _bin_bounds()
    inb = ((conf > lowers) & (conf <= uppers)).astype(jnp.float32)

    cnt = jnp.sum(inb, axis=0, keepdims=True)
    asum = jnp.sum(inb * correct, axis=0, keepdims=True)
    csum = jnp.sum(inb * conf, axis=0, keepdims=True)

    acc_ref[0:1, :] += cnt
    acc_ref[1:2, :] += asum
    acc_ref[2:3, :] += csum

    @pl.when(i == nblocks - 1)
    def _finish():
        cnt_v = acc_ref[0:1, :]
        asum_v = acc_ref[1:2, :]
        csum_v = acc_ref[2:3, :]
        prop = cnt_v / jnp.float32(n_total)
        safe = jnp.maximum(cnt_v, 1.0)
        acc_in = asum_v / safe
        conf_in = csum_v / safe
        per_bin = jnp.where(cnt_v > 0.0, jnp.abs(conf_in - acc_in) * prop, 0.0)
        out_ref[...] = jnp.sum(per_bin).reshape(1, 1)


def kernel(logits, labels):
    n, c = logits.shape
    block_rows = 4000
    nblocks = n // block_rows
    labels2d = labels.reshape(n, 1).astype(jnp.int32)

    import functools
    out = pl.pallas_call(
        functools.partial(_ece_kernel, nblocks=nblocks, n_total=n),
        grid=(nblocks,),
        in_specs=[
            pl.BlockSpec((block_rows, c), lambda i: (i, 0)),
            pl.BlockSpec((block_rows, 1), lambda i: (i, 0)),
        ],
        out_specs=pl.BlockSpec((1, 1), lambda i: (0, 0)),
        out_shape=jax.ShapeDtypeStruct((1, 1), jnp.float32),
        scratch_shapes=[pltpu.VMEM((8, 128), jnp.float32)],
    )(logits, labels2d)
    return out.reshape(1)


# speedup vs baseline: 1.1059x; 1.1059x over previous
import functools
import jax
import jax.numpy as jnp
from jax import lax
from jax.experimental import pallas as pl
from jax.experimental.pallas import tpu as pltpu

N_BINS = 15
NSTREAM = 4


def _recip_bounds():
    lane = lax.broadcasted_iota(jnp.int32, (1, 128), 1)
    lane_f = lane.astype(jnp.float32)
    b = jnp.where(lane == 0, jnp.float32(3.0e38), 15.0 / lane_f)
    return jnp.where(lane <= N_BINS, b, jnp.float32(-1.0))


def _colsel(nblocks, j):
    row = lax.broadcasted_iota(jnp.int32, (nblocks, 1), 0)
    return (row == j).astype(jnp.float32)


def _ece_kernel(l0, l1, l2, l3, labT_ref, out_ref, acc_ref, *, nsteps, n_total):
    i = pl.program_id(0)

    @pl.when(i == 0)
    def _init():
        acc_ref[...] = jnp.zeros_like(acc_ref)

    bounds = _recip_bounds()
    nblocks = NSTREAM * nsteps
    for k, ref in enumerate((l0, l1, l2, l3)):
        x = ref[...]
        r = x.shape[0]
        m = jnp.max(x, axis=1, keepdims=True)
        e = jnp.exp(x - m)
        ones_c = jnp.ones((x.shape[1], 1), jnp.float32)
        s = lax.dot_general(e, ones_c, (((1,), (0,)), ((), ())),
                            preferred_element_type=jnp.float32)
        conf = 1.0 / s
        pred = jnp.argmax(x, axis=1).reshape(r, 1)
        lab = lax.dot_general(labT_ref[...], _colsel(nblocks, NSTREAM * i + k),
                              (((1,), (0,)), ((), ())),
                              preferred_element_type=jnp.float32)
        acc = (pred.astype(jnp.float32) == lab).astype(jnp.float32)

        gt = (s < bounds).astype(jnp.float32)
        ones_r = jnp.ones((r, 1), jnp.float32)
        cnt = lax.dot_general(gt, ones_r, (((0,), (0,)), ((), ())),
                              preferred_element_type=jnp.float32)
        asum = lax.dot_general(gt, acc, (((0,), (0,)), ((), ())),
                               preferred_element_type=jnp.float32)
        csum = lax.dot_general(gt, conf, (((0,), (0,)), ((), ())),
                               preferred_element_type=jnp.float32)
        acc_ref[:, 0:1] += cnt
        acc_ref[:, 1:2] += asum
        acc_ref[:, 2:3] += csum

    @pl.when(i == nsteps - 1)
    def _finish():
        cum = acc_ref[...]
        cnt = cum[0:N_BINS, 0:1] - cum[1:N_BINS + 1, 0:1]
        asum = cum[0:N_BINS, 1:2] - cum[1:N_BINS + 1, 1:2]
        csum = cum[0:N_BINS, 2:3] - cum[1:N_BINS + 1, 2:3]
        prop = cnt / jnp.float32(n_total)
        safe = jnp.maximum(cnt, 1.0)
        per_bin = jnp.where(cnt > 0.0, jnp.abs(csum / safe - asum / safe) * prop, 0.0)
        out_ref[...] = jnp.sum(per_bin).reshape(1, 1)


def kernel(logits, labels):
    n, c = logits.shape
    block_rows = 2000
    nblocks = n // block_rows
    nsteps = nblocks // NSTREAM
    labT = labels.astype(jnp.float32).reshape(nblocks, block_rows).T

    out = pl.pallas_call(
        functools.partial(_ece_kernel, nsteps=nsteps, n_total=n),
        grid=(nsteps,),
        in_specs=[
            pl.BlockSpec((block_rows, c), lambda i: (NSTREAM * i, 0)),
            pl.BlockSpec((block_rows, c), lambda i: (NSTREAM * i + 1, 0)),
            pl.BlockSpec((block_rows, c), lambda i: (NSTREAM * i + 2, 0)),
            pl.BlockSpec((block_rows, c), lambda i: (NSTREAM * i + 3, 0)),
            pl.BlockSpec((block_rows, nblocks), lambda i: (0, 0)),
        ],
        out_specs=pl.BlockSpec((1, 1), lambda i: (0, 0)),
        out_shape=jax.ShapeDtypeStruct((1, 1), jnp.float32),
        scratch_shapes=[pltpu.VMEM((128, 8), jnp.float32)],
    )(logits, logits, logits, logits, labT)
    return out.reshape(1)


# speedup vs baseline: 1.4672x; 1.3267x over previous
import functools
import jax
import jax.numpy as jnp
from jax import lax
from jax.experimental import pallas as pl
from jax.experimental.pallas import tpu as pltpu

N_BINS = 15
NSTREAM = 4


def _recip_bounds():
    lane = lax.broadcasted_iota(jnp.int32, (1, 128), 1)
    lane_f = lane.astype(jnp.float32)
    b = jnp.where(lane == 0, jnp.float32(3.0e38), 15.0 / lane_f)
    return jnp.where(lane <= N_BINS, b, jnp.float32(-1.0))


def _colsel(nblocks, j):
    row = lax.broadcasted_iota(jnp.int32, (nblocks, 1), 0)
    return (row == j).astype(jnp.float32)


def _ece_kernel(l0, l1, l2, l3, labT_ref, out_ref, acc_ref, *, nsteps, n_total):
    i = pl.program_id(0)

    @pl.when(i == 0)
    def _init():
        acc_ref[...] = jnp.zeros_like(acc_ref)

    bounds = _recip_bounds()
    nblocks = NSTREAM * nsteps
    for k, ref in enumerate((l0, l1, l2, l3)):
        x = ref[...]
        r = x.shape[0]
        m = jnp.max(x, axis=1, keepdims=True)
        e = jnp.exp(x - m)
        s = jnp.sum(e, axis=1, keepdims=True)
        conf = 1.0 / s
        pred = jnp.argmax(x, axis=1).reshape(r, 1)
        lab = lax.dot_general(labT_ref[...], _colsel(nblocks, NSTREAM * i + k),
                              (((1,), (0,)), ((), ())),
                              preferred_element_type=jnp.float32)
        acc = (pred.astype(jnp.float32) == lab).astype(jnp.float32)

        gt = (s < bounds).astype(jnp.float32)
        acc_ref[0:1, :] += jnp.sum(gt, axis=0, keepdims=True)
        acc_ref[1:2, :] += jnp.sum(gt * acc, axis=0, keepdims=True)
        acc_ref[2:3, :] += jnp.sum(gt * conf, axis=0, keepdims=True)

    @pl.when(i == nsteps - 1)
    def _finish():
        cum = acc_ref[...]
        cnt = cum[0:1, 0:N_BINS] - cum[0:1, 1:N_BINS + 1]
        asum = cum[1:2, 0:N_BINS] - cum[1:2, 1:N_BINS + 1]
        csum = cum[2:3, 0:N_BINS] - cum[2:3, 1:N_BINS + 1]
        prop = cnt / jnp.float32(n_total)
        safe = jnp.maximum(cnt, 1.0)
        per_bin = jnp.where(cnt > 0.0, jnp.abs(csum / safe - asum / safe) * prop, 0.0)
        out_ref[...] = jnp.sum(per_bin).reshape(1, 1)


def kernel(logits, labels):
    n, c = logits.shape
    block_rows = 2000
    nblocks = n // block_rows
    nsteps = nblocks // NSTREAM
    labT = labels.astype(jnp.float32).reshape(nblocks, block_rows).T

    out = pl.pallas_call(
        functools.partial(_ece_kernel, nsteps=nsteps, n_total=n),
        grid=(nsteps,),
        in_specs=[
            pl.BlockSpec((block_rows, c), lambda i: (NSTREAM * i, 0)),
            pl.BlockSpec((block_rows, c), lambda i: (NSTREAM * i + 1, 0)),
            pl.BlockSpec((block_rows, c), lambda i: (NSTREAM * i + 2, 0)),
            pl.BlockSpec((block_rows, c), lambda i: (NSTREAM * i + 3, 0)),
            pl.BlockSpec((block_rows, nblocks), lambda i: (0, 0)),
        ],
        out_specs=pl.BlockSpec((1, 1), lambda i: (0, 0)),
        out_shape=jax.ShapeDtypeStruct((1, 1), jnp.float32),
        scratch_shapes=[pltpu.VMEM((8, 128), jnp.float32)],
    )(logits, logits, logits, logits, labT)
    return out.reshape(1)
